# baseline (speedup 1.0000x reference)
import functools

import jax
import jax.numpy as jnp
from jax import lax
from jax.experimental import pallas as pl
from jax.experimental.pallas import tpu as pltpu
from jax.experimental.pallas import tpu_sc as plsc

H = 16
KM = 4
DIM = 2
IDX_CHUNK = 128
PW = 80
OW = 128


def _sc_gather(tbl, r1, r2):
    n = r1.shape[0]
    info = plsc.get_sparse_core_info()
    nc, ns = info.num_cores, info.num_subcores
    nw = nc * ns
    bpw = n // nw
    nch = bpw // IDX_CHUNK

    mesh = plsc.VectorSubcoreMesh(core_axis_name="c", subcore_axis_name="s")
    f32 = jnp.float32
    out_type = [
        jax.ShapeDtypeStruct((n, OW), f32),
        jax.ShapeDtypeStruct((n, OW), f32),
    ]
    scratch_types = [
        pltpu.VMEM((bpw,), jnp.int32),
        pltpu.VMEM((bpw,), jnp.int32),
        pltpu.VMEM((bpw, PW), f32),
        pltpu.VMEM((bpw, PW), f32),
        pltpu.SemaphoreType.DMA,
    ]

    @functools.partial(pl.kernel, mesh=mesh, out_type=out_type,
                       scratch_types=scratch_types,
                       compiler_params=pltpu.CompilerParams(
                           use_tc_tiling_on_sc=False))
    def k(tref, r1h, r2h, oa, ob, i1, i2, bufa, bufb, sem):
        wid = lax.axis_index("s") * nc + lax.axis_index("c")
        base = wid * bpw
        pltpu.sync_copy(r1h.at[pl.ds(base, bpw)], i1)
        pltpu.sync_copy(r2h.at[pl.ds(base, bpw)], i2)
        handles = []
        for idxv, buf in ((i1, bufa), (i2, bufb)):
            for c in range(nch):
                handles.append(pltpu.async_copy(
                    tref.at[idxv.at[pl.ds(c * IDX_CHUNK, IDX_CHUNK)]],
                    buf.at[pl.ds(c * IDX_CHUNK, IDX_CHUNK), :],
                    sem))
        for hdl in handles:
            hdl.wait()
        pltpu.sync_copy(bufa, oa.at[pl.ds(base, bpw), pl.ds(0, PW)])
        pltpu.sync_copy(bufb, ob.at[pl.ds(base, bpw), pl.ds(0, PW)])

    return k(tbl, r1, r2)


def _tc_body(ga_r, gb_r, out_r):
    def parts(g):
        t = g.T
        return (t[:H], t[H:2 * H],
                t[2 * H:3 * H], t[3 * H:4 * H],
                t[64:72], t[72:76], t[76:77])

    a0, a1, ab1, aw2, amx, amm, ab2 = parts(ga_r[...])
    b0, b1v, bb1, bw2, bmx, bmm, bb2 = parts(gb_r[...])

    def side(w1d0, w1d1, b1f, w2f, b2f, mxm, mmm):
        u = jnp.zeros_like(b1f)
        for k in range(KM):
            z = (w1d0 * mxm[2 * k:2 * k + 1]
                 + w1d1 * mxm[2 * k + 1:2 * k + 2] + b1f)
            u = u + mmm[k:k + 1] * jnp.tanh(z)
        s = jnp.sum(u * w2f, axis=0)
        return s + b2f[0] * jnp.sum(mmm, axis=0)

    s1 = side(a0, a1, ab1, aw2, ab2, bmx, bmm)
    s2 = side(b0, b1v, bb1, bw2, bb2, amx, amm)
    out_r[...] = s1 + s2


def _tc_compute(ga, gb):
    n = ga.shape[0]
    bt = 2048
    return pl.pallas_call(
        _tc_body,
        grid=(n // bt,),
        in_specs=[pl.BlockSpec((bt, OW), lambda i: (i, 0)),
                  pl.BlockSpec((bt, OW), lambda i: (i, 0))],
        out_specs=pl.BlockSpec((bt,), lambda i: (i,)),
        out_shape=jax.ShapeDtypeStruct((n,), jnp.float32),
    )(ga, gb)


def kernel(pairs, W1, b1, w2, b2, mx, mm):
    v = W1.shape[0]
    r1 = pairs[:, 0].astype(jnp.int32)
    r2 = pairs[:, 1].astype(jnp.int32)
    stack = jnp.concatenate(
        [W1.transpose(2, 1, 0).reshape(2 * H, v),
         b1.T, w2.T,
         mx.transpose(1, 2, 0).reshape(KM * DIM, v),
         mm.T, b2[None, :],
         jnp.zeros((PW - 4 * H - KM * DIM - KM - 1, v), jnp.float32)],
        axis=0)
    stack = jax.lax.optimization_barrier(stack)
    tbl = stack.T
    ga, gb = _sc_gather(tbl, r1, r2)
    return _tc_compute(ga, gb)


# speedup vs baseline: 2.1145x; 2.1145x over previous
import functools

import jax
import jax.numpy as jnp
from jax import lax
from jax.experimental import pallas as pl
from jax.experimental.pallas import tpu as pltpu
from jax.experimental.pallas import tpu_sc as plsc

H = 16
KM = 4
DIM = 2
IDX_CHUNK = 128
OW = 128


def _sc_gather(w1f, b1, w2, me, r1, r2):
    n = r1.shape[0]
    info = plsc.get_sparse_core_info()
    nc, ns = info.num_cores, info.num_subcores
    nw = nc * ns
    bpw = n // nw
    nch = bpw // IDX_CHUNK

    mesh = plsc.VectorSubcoreMesh(core_axis_name="c", subcore_axis_name="s")
    f32 = jnp.float32
    out_type = [
        jax.ShapeDtypeStruct((n, OW), f32),
        jax.ShapeDtypeStruct((n, OW), f32),
    ]
    scratch_types = [
        pltpu.VMEM((bpw,), jnp.int32),
        pltpu.VMEM((bpw,), jnp.int32),
        pltpu.VMEM((bpw, 2 * H), f32),
        pltpu.VMEM((bpw, H), f32),
        pltpu.VMEM((bpw, H), f32),
        pltpu.VMEM((bpw, H), f32),
        pltpu.VMEM((bpw, 2 * H), f32),
        pltpu.VMEM((bpw, H), f32),
        pltpu.VMEM((bpw, H), f32),
        pltpu.VMEM((bpw, H), f32),
        pltpu.SemaphoreType.DMA,
    ]

    @functools.partial(pl.kernel, mesh=mesh, out_type=out_type,
                       scratch_types=scratch_types,
                       compiler_params=pltpu.CompilerParams(
                           use_tc_tiling_on_sc=False))
    def k(tw1, tb1, tw2, tme, r1h, r2h, oa, ob,
          i1, i2, bw1a, bb1a, bw2a, bmea, bw1b, bb1b, bw2b, bmeb, sem):
        wid = lax.axis_index("s") * nc + lax.axis_index("c")
        base = wid * bpw
        pltpu.sync_copy(r1h.at[pl.ds(base, bpw)], i1)
        pltpu.sync_copy(r2h.at[pl.ds(base, bpw)], i2)
        handles = []
        for idxv, bufs in ((i1, (bw1a, bb1a, bw2a, bmea)),
                           (i2, (bw1b, bb1b, bw2b, bmeb))):
            for tbl, buf in zip((tw1, tb1, tw2, tme), bufs):
                for c in range(nch):
                    handles.append(pltpu.async_copy(
                        tbl.at[idxv.at[pl.ds(c * IDX_CHUNK, IDX_CHUNK)]],
                        buf.at[pl.ds(c * IDX_CHUNK, IDX_CHUNK), :],
                        sem))
        for hdl in handles:
            hdl.wait()
        for out, bufs in ((oa, (bw1a, bb1a, bw2a, bmea)),
                          (ob, (bw1b, bb1b, bw2b, bmeb))):
            col = 0
            for buf in bufs:
                w = buf.shape[1]
                pltpu.sync_copy(buf, out.at[pl.ds(base, bpw), pl.ds(col, w)])
                col += w

    return k(w1f, b1, w2, me, r1, r2)


def _tc_body(ga_r, gb_r, out_r):
    def parts(g):
        t = g.T
        return (t[:H], t[H:2 * H],
                t[2 * H:3 * H], t[3 * H:4 * H],
                t[64:72], t[72:76], t[76:77])

    a0, a1, ab1, aw2, amx, amm, ab2 = parts(ga_r[...])
    b0, b1v, bb1, bw2, bmx, bmm, bb2 = parts(gb_r[...])

    def side(w1d0, w1d1, b1f, w2f, b2f, mxm, mmm):
        u = jnp.zeros_like(b1f)
        for k in range(KM):
            z = (w1d0 * mxm[2 * k:2 * k + 1]
                 + w1d1 * mxm[2 * k + 1:2 * k + 2] + b1f)
            u = u + mmm[k:k + 1] * jnp.tanh(z)
        s = jnp.sum(u * w2f, axis=0)
        return s + b2f[0] * jnp.sum(mmm, axis=0)

    s1 = side(a0, a1, ab1, aw2, ab2, bmx, bmm)
    s2 = side(b0, b1v, bb1, bw2, bb2, amx, amm)
    out_r[...] = s1 + s2


def _tc_compute(ga, gb):
    n = ga.shape[0]
    bt = 4096
    return pl.pallas_call(
        _tc_body,
        grid=(n // bt,),
        in_specs=[pl.BlockSpec((bt, OW), lambda i: (i, 0)),
                  pl.BlockSpec((bt, OW), lambda i: (i, 0))],
        out_specs=pl.BlockSpec((bt,), lambda i: (i,)),
        out_shape=jax.ShapeDtypeStruct((n,), jnp.float32),
    )(ga, gb)


def kernel(pairs, W1, b1, w2, b2, mx, mm):
    v = W1.shape[0]
    r1 = pairs[:, 0].astype(jnp.int32)
    r2 = pairs[:, 1].astype(jnp.int32)
    w1f = jnp.swapaxes(W1, 1, 2).reshape(v, 2 * H)
    me = jnp.concatenate(
        [mx.reshape(v, KM * DIM), mm, b2[:, None],
         jnp.zeros((v, H - KM * DIM - KM - 1), jnp.float32)], axis=1)
    ga, gb = _sc_gather(w1f, b1, w2, me, r1, r2)
    return _tc_compute(ga, gb)


# speedup vs baseline: 2.1580x; 1.0206x over previous
import functools

import jax
import jax.numpy as jnp
from jax import lax
from jax.experimental import pallas as pl
from jax.experimental.pallas import tpu as pltpu
from jax.experimental.pallas import tpu_sc as plsc

H = 16
KM = 4
DIM = 2
IDX_CHUNK = 128
OW = 128


def _sc_gather(w1f, bwme, r1, r2):
    n = r1.shape[0]
    info = plsc.get_sparse_core_info()
    nc, ns = info.num_cores, info.num_subcores
    nw = nc * ns
    bpw = n // nw
    nch = bpw // IDX_CHUNK

    mesh = plsc.VectorSubcoreMesh(core_axis_name="c", subcore_axis_name="s")
    f32 = jnp.float32
    out_type = [
        jax.ShapeDtypeStruct((n, OW), f32),
        jax.ShapeDtypeStruct((n, OW), f32),
    ]
    scratch_types = [
        pltpu.VMEM((bpw,), jnp.int32),
        pltpu.VMEM((bpw,), jnp.int32),
        pltpu.VMEM((bpw, 2 * H), f32),
        pltpu.VMEM((bpw, 3 * H), f32),
        pltpu.VMEM((bpw, 2 * H), f32),
        pltpu.VMEM((bpw, 3 * H), f32),
        pltpu.SemaphoreType.DMA,
    ]

    @functools.partial(pl.kernel, mesh=mesh, out_type=out_type,
                       scratch_types=scratch_types,
                       compiler_params=pltpu.CompilerParams(
                           use_tc_tiling_on_sc=False))
    def k(tw1, tbw, r1h, r2h, oa, ob,
          i1, i2, bw1a, bwmea, bw1b, bwmeb, sem):
        wid = lax.axis_index("s") * nc + lax.axis_index("c")
        base = wid * bpw
        pltpu.sync_copy(r1h.at[pl.ds(base, bpw)], i1)
        pltpu.sync_copy(r2h.at[pl.ds(base, bpw)], i2)
        handles = []
        for idxv, bufs in ((i1, (bw1a, bwmea)),
                           (i2, (bw1b, bwmeb))):
            for tbl, buf in zip((tw1, tbw), bufs):
                for c in range(nch):
                    handles.append(pltpu.async_copy(
                        tbl.at[idxv.at[pl.ds(c * IDX_CHUNK, IDX_CHUNK)]],
                        buf.at[pl.ds(c * IDX_CHUNK, IDX_CHUNK), :],
                        sem))
        for hdl in handles:
            hdl.wait()
        for out, bufs in ((oa, (bw1a, bwmea)),
                          (ob, (bw1b, bwmeb))):
            col = 0
            for buf in bufs:
                w = buf.shape[1]
                pltpu.sync_copy(buf, out.at[pl.ds(base, bpw), pl.ds(col, w)])
                col += w

    return k(w1f, bwme, r1, r2)


def _tc_body(ga_r, gb_r, out_r):
    def parts(g):
        t = g.T
        return (t[:H], t[H:2 * H],
                t[2 * H:3 * H], t[3 * H:4 * H],
                t[64:72], t[72:76], t[76:77])

    a0, a1, ab1, aw2, amx, amm, ab2 = parts(ga_r[...])
    b0, b1v, bb1, bw2, bmx, bmm, bb2 = parts(gb_r[...])

    def side(w1d0, w1d1, b1f, w2f, b2f, mxm, mmm):
        u = jnp.zeros_like(b1f)
        for k in range(KM):
            z = (w1d0 * mxm[2 * k:2 * k + 1]
                 + w1d1 * mxm[2 * k + 1:2 * k + 2] + b1f)
            u = u + mmm[k:k + 1] * jnp.tanh(z)
        s = jnp.sum(u * w2f, axis=0)
        return s + b2f[0] * jnp.sum(mmm, axis=0)

    s1 = side(a0, a1, ab1, aw2, ab2, bmx, bmm)
    s2 = side(b0, b1v, bb1, bw2, bb2, amx, amm)
    out_r[...] = s1 + s2


def _tc_compute(ga, gb):
    n = ga.shape[0]
    bt = 4096
    return pl.pallas_call(
        _tc_body,
        grid=(n // bt,),
        in_specs=[pl.BlockSpec((bt, OW), lambda i: (i, 0)),
                  pl.BlockSpec((bt, OW), lambda i: (i, 0))],
        out_specs=pl.BlockSpec((bt,), lambda i: (i,)),
        out_shape=jax.ShapeDtypeStruct((n,), jnp.float32),
    )(ga, gb)


def kernel(pairs, W1, b1, w2, b2, mx, mm):
    v = W1.shape[0]
    r1 = pairs[:, 0].astype(jnp.int32)
    r2 = pairs[:, 1].astype(jnp.int32)
    w1f = jnp.swapaxes(W1, 1, 2).reshape(v, 2 * H)
    bwme = jnp.concatenate(
        [b1, w2, mx.reshape(v, KM * DIM), mm, b2[:, None],
         jnp.zeros((v, H - KM * DIM - KM - 1), jnp.float32)], axis=1)
    ga, gb = _sc_gather(w1f, bwme, r1, r2)
    return _tc_compute(ga, gb)
